# Initial kernel scaffold; baseline (speedup 1.0000x reference)
import functools

import jax
import jax.numpy as jnp
from jax import lax
from jax.experimental import pallas as pl
from jax.experimental.pallas import tpu as pltpu
from jax.experimental.pallas import tpu_sc as plsc

_VOCAB = 100000
_SEQ = 200
_DIM = 128
_BATCH = 1024
_NC = 2
_NS = 16
_NW = _NC * _NS
_ROWS = _BATCH * _SEQ
_RPW = _ROWS // _NW
_CH = _SEQ
_NCHUNK = _RPW // _CH
_LANES = 16


def _body(table_hbm, idx_hbm, pos_hbm, out_hbm, pos_v, idx_v, rows_v, gsem):
    wid = lax.axis_index("s") * _NC + lax.axis_index("c")
    base = wid * _RPW
    pltpu.sync_copy(pos_hbm, pos_v)

    def chunk(c, carry):
        gbase = base + c * _CH
        pltpu.sync_copy(idx_hbm.at[pl.ds(gbase, _CH)], idx_v)
        cp0 = pltpu.async_copy(
            table_hbm.at[idx_v.at[pl.ds(0, 128)]], rows_v.at[pl.ds(0, 128)], gsem)
        cp1 = pltpu.async_copy(
            table_hbm.at[idx_v.at[pl.ds(128, _CH - 128)]],
            rows_v.at[pl.ds(128, _CH - 128)], gsem)
        cp0.wait()
        cp1.wait()

        def row(r, carry2):
            for j in range(_DIM // _LANES):
                sl = pl.ds(j * _LANES, _LANES)
                plsc.addupdate(rows_v.at[r, sl], pos_v[r, sl])
            return carry2

        lax.fori_loop(0, _CH, row, 0)
        pltpu.sync_copy(rows_v, out_hbm.at[pl.ds(gbase, _CH)])
        return carry

    lax.fori_loop(0, _NCHUNK, chunk, 0)


def kernel(input, W_vocab, W_pos):
    idx = input.reshape(_ROWS).astype(jnp.int32)
    mesh = plsc.VectorSubcoreMesh(
        core_axis_name="c", subcore_axis_name="s",
        num_cores=_NC, num_subcores=_NS)
    out = pl.kernel(
        _body,
        out_type=jax.ShapeDtypeStruct((_ROWS, _DIM), jnp.float32),
        mesh=mesh,
        scratch_types=[
            pltpu.VMEM((_SEQ, _DIM), jnp.float32),
            pltpu.VMEM((_CH,), jnp.int32),
            pltpu.VMEM((_CH, _DIM), jnp.float32),
            pltpu.SemaphoreType.DMA,
        ],
    )(W_vocab, idx, W_pos)
    return out.reshape(_BATCH, _SEQ, _DIM)


# speedup vs baseline: 3.7260x; 3.7260x over previous
import functools

import jax
import jax.numpy as jnp
from jax import lax
from jax.experimental import pallas as pl
from jax.experimental.pallas import tpu as pltpu
from jax.experimental.pallas import tpu_sc as plsc

_VOCAB = 100000
_SEQ = 200
_DIM = 128
_BATCH = 1024
_NC = 2
_NS = 16
_NW = _NC * _NS
_ROWS = _BATCH * _SEQ
_RPW = _ROWS // _NW
_CH = _SEQ
_NCHUNK = _RPW // _CH
_LANES = 16


def _body(table_hbm, idx_hbm, pos_hbm, out_hbm, pos_v, idx_a, idx_b, rows_v,
          sem_a, sem_b):
    wid = lax.axis_index("s") * _NC + lax.axis_index("c")
    base = wid * _RPW
    pltpu.sync_copy(pos_hbm, pos_v)

    def chunk(c, carry):
        gbase = base + c * _CH
        pltpu.sync_copy(idx_hbm.at[pl.ds(gbase, 128)], idx_a)
        pltpu.sync_copy(idx_hbm.at[pl.ds(gbase + 128, _CH - 128)], idx_b)
        cp0 = pltpu.async_copy(
            table_hbm.at[idx_a], rows_v.at[pl.ds(0, 128)], sem_a)
        cp1 = pltpu.async_copy(
            table_hbm.at[idx_b], rows_v.at[pl.ds(128, _CH - 128)], sem_b)
        cp0.wait()
        cp1.wait()

        def row(r, carry2):
            for j in range(_DIM // _LANES):
                sl = pl.ds(j * _LANES, _LANES)
                plsc.addupdate(rows_v.at[r, sl], pos_v[r, sl])
            return carry2

        lax.fori_loop(0, _CH, row, 0)
        pltpu.sync_copy(rows_v, out_hbm.at[pl.ds(gbase, _CH)])
        return carry

    lax.fori_loop(0, _NCHUNK, chunk, 0)


def kernel(input, W_vocab, W_pos):
    idx = input.reshape(_ROWS).astype(jnp.int32)
    mesh = plsc.VectorSubcoreMesh(
        core_axis_name="c", subcore_axis_name="s",
        num_cores=_NC, num_subcores=_NS)
    out = pl.kernel(
        _body,
        out_type=jax.ShapeDtypeStruct((_ROWS, _DIM), jnp.float32),
        mesh=mesh,
        scratch_types=[
            pltpu.VMEM((_SEQ, _DIM), jnp.float32),
            pltpu.VMEM((128,), jnp.int32),
            pltpu.VMEM((_CH - 128,), jnp.int32),
            pltpu.VMEM((_CH, _DIM), jnp.float32),
            pltpu.SemaphoreType.DMA,
            pltpu.SemaphoreType.DMA,
        ],
    )(W_vocab, idx, W_pos)
    return out.reshape(_BATCH, _SEQ, _DIM)
